# Initial kernel scaffold; baseline (speedup 1.0000x reference)
import jax
import jax.numpy as jnp
from jax.experimental import pallas as pl


def kernel(x, edge_index, batch, params):
    raise NotImplementedError("write your pallas kernel here")


# speedup vs baseline: 1.1141x; 1.1141x over previous
import jax
import jax.numpy as jnp
from jax.experimental import pallas as pl

_N = 10000
_G = 64
_EPS = 1e-5


def _bn(x, g, b):
    mu = jnp.mean(x, axis=0)
    var = jnp.mean((x - mu) ** 2, axis=0)
    return g * (x - mu) / jnp.sqrt(var + _EPS) + b


def _gat(x, src, dst, W, a_s, a_d, b, H, C):
    h = (x @ W).reshape(_N, H, C)
    s = (h * a_s).sum(-1)
    d = (h * a_d).sum(-1)
    M = jax.nn.leaky_relu(jnp.max(s, axis=0) + jnp.max(d, axis=0), 0.2)
    e = jax.nn.leaky_relu(s[src] + d[dst], 0.2)
    ee = jnp.exp(e - M[None, :])
    den = jax.ops.segment_sum(ee, dst, num_segments=_N)
    num = jax.ops.segment_sum(h[src] * ee[:, :, None], dst, num_segments=_N)
    ee_self = jnp.exp(jax.nn.leaky_relu(s + d, 0.2) - M[None, :])
    den = den + ee_self
    num = num + h * ee_self[:, :, None]
    out = num / (den + 1e-16)[:, :, None]
    return out.reshape(_N, H * C) + b


def _head_body(pooled_ref, wc1_ref, bc1_ref, gc1_ref, bec1_ref,
               wc2_ref, bc2_ref, gc2_ref, bec2_ref, wc3_ref, bc3_ref, out_ref):
    pooled = pooled_ref[...]
    c = pooled @ wc1_ref[...] + bc1_ref[...]
    mu = jnp.mean(c, axis=0, keepdims=True)
    var = jnp.mean((c - mu) ** 2, axis=0, keepdims=True)
    c = gc1_ref[...] * (c - mu) * jax.lax.rsqrt(var + _EPS) + bec1_ref[...]
    c = jax.nn.relu(c)
    c = c @ wc2_ref[...] + bc2_ref[...]
    mu = jnp.mean(c, axis=0, keepdims=True)
    var = jnp.mean((c - mu) ** 2, axis=0, keepdims=True)
    c = gc2_ref[...] * (c - mu) * jax.lax.rsqrt(var + _EPS) + bec2_ref[...]
    c = jax.nn.relu(c)
    out_ref[...] = c @ wc3_ref[...] + bc3_ref[...]


def _head(pooled, p):
    args = (pooled,
            p['Wc1'], p['bc1'][None, :], p['gc1'][None, :], p['bec1'][None, :],
            p['Wc2'], p['bc2'][None, :], p['gc2'][None, :], p['bec2'][None, :],
            p['Wc3'], p['bc3'][None, :])
    return pl.pallas_call(
        _head_body,
        out_shape=jax.ShapeDtypeStruct((_G, 2), jnp.float32),
    )(*args)


def kernel(x, edge_index, batch, params):
    p = params
    src, dst = edge_index[0], edge_index[1]
    h = jax.nn.relu(_bn(x @ p['W0'] + p['b0'], p['g0'], p['be0']))
    h = jax.nn.relu(_bn(_gat(h, src, dst, p['W1'], p['as1'], p['ad1'], p['b1'], 8, 32), p['g1'], p['be1']))
    h = jax.nn.relu(_bn(_gat(h, src, dst, p['W2'], p['as2'], p['ad2'], p['b2'], 8, 16), p['g2'], p['be2']))
    h = jax.nn.relu(_bn(_gat(h, src, dst, p['W3'], p['as3'], p['ad3'], p['b3'], 8, 8), p['g3'], p['be3']))
    gate = (jax.nn.relu(_bn(h @ p['Wg1'] + p['bg1'], p['gg'], p['beg'])) @ p['Wg2'] + p['bg2'])[:, 0]
    ge = jnp.exp(gate - jnp.max(gate))
    B1h = (batch[:, None] == jnp.arange(_G, dtype=batch.dtype)[None, :]).astype(jnp.float32)
    hi = jax.lax.Precision.HIGHEST
    gden = jnp.matmul(B1h.T, ge, precision=hi)
    w = (ge / (jnp.matmul(B1h, gden, precision=hi) + 1e-16))[:, None]
    pooled = jnp.matmul(B1h.T, w * h, precision=hi)
    return _head(pooled, p)


# speedup vs baseline: 41.7214x; 37.4479x over previous
import functools

import jax
import jax.numpy as jnp
from jax import lax
from jax.experimental import pallas as pl
from jax.experimental.pallas import tpu as pltpu
from jax.experimental.pallas import tpu_sc as plsc

_N = 10000
_E = 320000
_G = 64
_EPS = 1e-5

_K = 128
_E_PAD = 323584
_NROWS = 10112
_RPT = _NROWS // 16


def _bn(x, g, b):
    mu = jnp.mean(x, axis=0)
    var = jnp.mean((x - mu) ** 2, axis=0)
    return g * (x - mu) / jnp.sqrt(var + _EPS) + b


def _make_edge_pass(Fc, logC, feature_split):
    R = 16 + Fc
    et = _E_PAD // 16 if feature_split else _E_PAD // 32
    nchunks = et // _K
    mesh = plsc.VectorSubcoreMesh(core_axis_name="c", subcore_axis_name="s")

    @functools.partial(
        pl.kernel, mesh=mesh,
        compiler_params=pltpu.CompilerParams(use_tc_tiling_on_sc=False),
        out_type=jax.ShapeDtypeStruct((2, _NROWS, R), jnp.float32),
        scratch_types=[
            pltpu.VMEM((_K,), jnp.int32),
            pltpu.VMEM((_K,), jnp.int32),
            pltpu.VMEM((_K, R), jnp.float32),
            pltpu.VMEM((_K, 16), jnp.float32),
            pltpu.VMEM((_K, R), jnp.float32),
            pltpu.VMEM((16,), jnp.float32),
            pltpu.VMEM_SHARED((_NROWS, R), jnp.float32),
            pltpu.SemaphoreType.DMA,
        ],
    )
    def edge_pass(src_p, dst_p, hs0, hs1, dtab0, dtab1, m16, zeros, out,
                  src_v, dst_v, hs_buf, d_buf, acc_buf, m_buf, spacc, sem):
        c = lax.axis_index("c")
        t = lax.axis_index("s")
        rows0 = pl.multiple_of(t * _RPT, 8)
        pltpu.sync_copy(zeros.at[pl.ds(rows0, _RPT), :],
                        spacc.at[pl.ds(rows0, _RPT), :])
        pltpu.sync_copy(m16.at[c], m_buf)
        plsc.subcore_barrier()
        mvec = m_buf[...]
        mask_lo = lax.iota(jnp.int32, 16) < 8
        if feature_split:
            ebase = t * et
        else:
            ebase = c * (_E_PAD // 2) + t * et

        def chunk(i, carry):
            base = pl.multiple_of(ebase + i * _K, 8)
            pltpu.sync_copy(src_p.at[pl.ds(base, _K)], src_v)
            pltpu.sync_copy(dst_p.at[pl.ds(base, _K)], dst_v)

            @pl.when(c == 0)
            def _():
                pltpu.async_copy(hs0.at[src_v], hs_buf, sem).wait()
                pltpu.async_copy(dtab0.at[dst_v], d_buf, sem).wait()

            @pl.when(c == 1)
            def _():
                pltpu.async_copy(hs1.at[src_v], hs_buf, sem).wait()
                pltpu.async_copy(dtab1.at[dst_v], d_buf, sem).wait()

            def edge(k, carry2):
                sv = hs_buf[k, pl.ds(0, 16)]
                e16 = sv + d_buf[k, pl.ds(0, 16)]
                e16 = jnp.where(e16 >= 0.0, e16, 0.2 * e16)
                ee = jnp.exp(e16 - mvec)
                acc_buf[k, pl.ds(0, 16)] = ee
                for j in range(Fc // 16):
                    hA = (j * 16) >> logC
                    hB = (j * 16 + 8) >> logC
                    if hA == hB:
                        mult = jnp.full((16,), ee[hA], jnp.float32)
                    else:
                        mult = jnp.where(mask_lo,
                                         jnp.full((16,), ee[hA], jnp.float32),
                                         jnp.full((16,), ee[hB], jnp.float32))
                    hv = hs_buf[k, pl.ds(16 + j * 16, 16)]
                    acc_buf[k, pl.ds(16 + j * 16, 16)] = hv * mult
                return carry2

            lax.fori_loop(0, _K, edge, 0)
            pltpu.sync_copy(acc_buf, spacc.at[dst_v], add=True)
            return carry

        lax.fori_loop(0, nchunks, chunk, 0)
        plsc.subcore_barrier()
        pltpu.sync_copy(spacc.at[pl.ds(rows0, _RPT), :],
                        out.at[c, pl.ds(rows0, _RPT), :])

    return edge_pass


_edge_l1 = _make_edge_pass(128, 5, True)
_edge_l2 = _make_edge_pass(128, 4, False)
_edge_l3 = _make_edge_pass(64, 3, False)


def _gat_sc(x, src_p, dst_p, W, a_s, a_d, b, H, C, edge_fn, feature_split):
    h = (x @ W).reshape(_N, H, C)
    hf = h.reshape(_N, H * C)
    s = (h * a_s).sum(-1)
    d = (h * a_d).sum(-1)
    M = jax.nn.leaky_relu(jnp.max(s, axis=0) + jnp.max(d, axis=0), 0.2)
    pad_hi = jnp.full((8,), 1e30, jnp.float32)
    pad8 = jnp.zeros((_N, 8), jnp.float32)
    F = H * C
    Fc = F if not feature_split else F // 2
    R = 16 + Fc

    def _dtab(dv):
        t = jnp.full((_NROWS, 16), -1e30, jnp.float32)
        return t.at[:_N, 0:8].set(dv).at[:_N, 8:16].set(0.0)

    if feature_split:
        s_r = jnp.roll(s, -4, axis=1)
        d_r = jnp.roll(d, -4, axis=1)
        M_r = jnp.roll(M, -4)
        hs0 = jnp.concatenate([s, pad8, hf[:, :Fc]], axis=1)
        hs1 = jnp.concatenate([s_r, pad8, hf[:, Fc:]], axis=1)
        dtab0 = _dtab(d)
        dtab1 = _dtab(d_r)
        m16 = jnp.stack([jnp.concatenate([M, pad_hi]),
                         jnp.concatenate([M_r, pad_hi])])
    else:
        hs0 = jnp.concatenate([s, pad8, hf], axis=1)
        hs1 = hs0
        dtab0 = _dtab(d)
        dtab1 = dtab0
        m16 = jnp.stack([jnp.concatenate([M, pad_hi])] * 2)
    zeros = jnp.zeros((_NROWS, R), jnp.float32)
    acc = edge_fn(src_p, dst_p, hs0, hs1, dtab0, dtab1, m16, zeros)
    if feature_split:
        den = acc[0, :_N, 0:8]
        num = jnp.concatenate([acc[0, :_N, 16:], acc[1, :_N, 16:]], axis=1)
    else:
        den = acc[0, :_N, 0:8] + acc[1, :_N, 0:8]
        num = acc[0, :_N, 16:] + acc[1, :_N, 16:]
    ee_self = jnp.exp(jax.nn.leaky_relu(s + d, 0.2) - M[None, :])
    den = den + ee_self
    num = num.reshape(_N, H, C) + h * ee_self[:, :, None]
    out = num / (den + 1e-16)[:, :, None]
    return out.reshape(_N, H * C) + b


def _head_body(pooled_ref, wc1_ref, bc1_ref, gc1_ref, bec1_ref,
               wc2_ref, bc2_ref, gc2_ref, bec2_ref, wc3_ref, bc3_ref, out_ref):
    pooled = pooled_ref[...]
    c = pooled @ wc1_ref[...] + bc1_ref[...]
    mu = jnp.mean(c, axis=0, keepdims=True)
    var = jnp.mean((c - mu) ** 2, axis=0, keepdims=True)
    c = gc1_ref[...] * (c - mu) * jax.lax.rsqrt(var + _EPS) + bec1_ref[...]
    c = jax.nn.relu(c)
    c = c @ wc2_ref[...] + bc2_ref[...]
    mu = jnp.mean(c, axis=0, keepdims=True)
    var = jnp.mean((c - mu) ** 2, axis=0, keepdims=True)
    c = gc2_ref[...] * (c - mu) * jax.lax.rsqrt(var + _EPS) + bec2_ref[...]
    c = jax.nn.relu(c)
    out_ref[...] = c @ wc3_ref[...] + bc3_ref[...]


def _head(pooled, p):
    args = (pooled,
            p['Wc1'], p['bc1'][None, :], p['gc1'][None, :], p['bec1'][None, :],
            p['Wc2'], p['bc2'][None, :], p['gc2'][None, :], p['bec2'][None, :],
            p['Wc3'], p['bc3'][None, :])
    return pl.pallas_call(
        _head_body,
        out_shape=jax.ShapeDtypeStruct((_G, 2), jnp.float32),
    )(*args)


def kernel(x, edge_index, batch, params):
    p = params
    pad = _E_PAD - _E
    src_p = jnp.concatenate([edge_index[0], jnp.zeros((pad,), jnp.int32)])
    dst_p = jnp.concatenate([edge_index[1], jnp.full((pad,), _N, jnp.int32)])
    h = jax.nn.relu(_bn(x @ p['W0'] + p['b0'], p['g0'], p['be0']))
    h = jax.nn.relu(_bn(_gat_sc(h, src_p, dst_p, p['W1'], p['as1'], p['ad1'], p['b1'], 8, 32, _edge_l1, True), p['g1'], p['be1']))
    h = jax.nn.relu(_bn(_gat_sc(h, src_p, dst_p, p['W2'], p['as2'], p['ad2'], p['b2'], 8, 16, _edge_l2, False), p['g2'], p['be2']))
    h = jax.nn.relu(_bn(_gat_sc(h, src_p, dst_p, p['W3'], p['as3'], p['ad3'], p['b3'], 8, 8, _edge_l3, False), p['g3'], p['be3']))
    gate = (jax.nn.relu(_bn(h @ p['Wg1'] + p['bg1'], p['gg'], p['beg'])) @ p['Wg2'] + p['bg2'])[:, 0]
    ge = jnp.exp(gate - jnp.max(gate))
    B1h = (batch[:, None] == jnp.arange(_G, dtype=batch.dtype)[None, :]).astype(jnp.float32)
    hi = jax.lax.Precision.HIGHEST
    gden = jnp.matmul(B1h.T, ge, precision=hi)
    w = (ge / (jnp.matmul(B1h, gden, precision=hi) + 1e-16))[:, None]
    pooled = jnp.matmul(B1h.T, w * h, precision=hi)
    return _head(pooled, p)


# speedup vs baseline: 71.7572x; 1.7199x over previous
import functools

import jax
import jax.numpy as jnp
from jax import lax
from jax.experimental import pallas as pl
from jax.experimental.pallas import tpu as pltpu
from jax.experimental.pallas import tpu_sc as plsc

_N = 10000
_E = 320000
_G = 64
_EPS = 1e-5

_K = 64
_E_PAD = 323584
_NROWS = 10112
_RPT = _NROWS // 16


def _bn(x, g, b):
    mu = jnp.mean(x, axis=0)
    var = jnp.mean((x - mu) ** 2, axis=0)
    return g * (x - mu) / jnp.sqrt(var + _EPS) + b


def _make_edge_pass(Fc, logC, feature_split):
    R = 16 + Fc
    et = _E_PAD // 16 if feature_split else _E_PAD // 32
    nchunks = et // _K
    npairs = nchunks // 2
    mesh = plsc.VectorSubcoreMesh(core_axis_name="c", subcore_axis_name="s")

    @functools.partial(
        pl.kernel, mesh=mesh,
        compiler_params=pltpu.CompilerParams(use_tc_tiling_on_sc=False),
        out_type=jax.ShapeDtypeStruct((2, _NROWS, R), jnp.float32),
        scratch_types=[
            pltpu.VMEM((2, _K), jnp.int32),
            pltpu.VMEM((2, _K), jnp.int32),
            pltpu.VMEM((_K, R), jnp.float32),
            pltpu.VMEM((_K, R), jnp.float32),
            pltpu.VMEM((_K, 16), jnp.float32),
            pltpu.VMEM((_K, 16), jnp.float32),
            pltpu.VMEM((_K, R), jnp.float32),
            pltpu.VMEM((16,), jnp.float32),
            pltpu.VMEM_SHARED((_NROWS, R), jnp.float32),
            pltpu.SemaphoreType.DMA,
            pltpu.SemaphoreType.DMA,
            pltpu.SemaphoreType.DMA,
            pltpu.SemaphoreType.DMA,
            pltpu.SemaphoreType.DMA,
            pltpu.SemaphoreType.DMA,
        ],
    )
    def edge_pass(ids2, hs0, hs1, dtab0, dtab1, m16, zeros, out,
                  ids_a, ids_b, hs_a, hs_b, d_a, d_b, acc_buf, m_buf, spacc,
                  isem_a, isem_b, gsem_a, gsem_b, dsem_a, dsem_b):
        c = lax.axis_index("c")
        t = lax.axis_index("s")
        rows0 = pl.multiple_of(t * _RPT, 8)
        pltpu.sync_copy(zeros.at[pl.ds(rows0, _RPT), :],
                        spacc.at[pl.ds(rows0, _RPT), :])
        pltpu.sync_copy(m16.at[c], m_buf)
        plsc.subcore_barrier()
        mvec = m_buf[...]
        mask_lo = lax.iota(jnp.int32, 16) < 8
        if feature_split:
            cbase = t * nchunks
        else:
            cbase = c * (_E_PAD // (2 * _K)) + t * nchunks

        A = (ids_a, hs_a, d_a, isem_a, gsem_a, dsem_a)
        B = (ids_b, hs_b, d_b, isem_b, gsem_b, dsem_b)

        def issue_ids(m, P):
            ids, _, _, isem, _, _ = P
            pltpu.async_copy(ids2.at[m], ids, isem)

        def wait_ids(P):
            ids, _, _, isem, _, _ = P
            pltpu.make_async_copy(ids2.at[0], ids, isem).wait()

        def issue_gather(P):
            ids, hsb, db, _, gsem, dsem = P

            @pl.when(c == 0)
            def _():
                pltpu.async_copy(hs0.at[ids.at[0]], hsb, gsem)
                pltpu.async_copy(dtab0.at[ids.at[1]], db, dsem)

            @pl.when(c == 1)
            def _():
                pltpu.async_copy(hs1.at[ids.at[0]], hsb, gsem)
                pltpu.async_copy(dtab1.at[ids.at[1]], db, dsem)

        def wait_gather(P):
            ids, hsb, db, _, gsem, dsem = P
            pltpu.make_async_copy(hs0.at[ids.at[0]], hsb, gsem).wait()
            pltpu.make_async_copy(dtab0.at[ids.at[1]], db, dsem).wait()

        def compute_scatter(P):
            ids, hsb, db, _, _, _ = P

            @plsc.parallel_loop(0, _K, unroll=2)
            def _edge(k):
                sv = hsb[k, pl.ds(0, 16)]
                e16 = sv + db[k, pl.ds(0, 16)]
                e16 = jnp.where(e16 >= 0.0, e16, 0.2 * e16)
                ee = jnp.exp(e16 - mvec)
                acc_buf[k, pl.ds(0, 16)] = ee
                for j in range(Fc // 16):
                    hA = (j * 16) >> logC
                    hB = (j * 16 + 8) >> logC
                    if hA == hB:
                        mult = jnp.full((16,), ee[hA], jnp.float32)
                    else:
                        mult = jnp.where(mask_lo,
                                         jnp.full((16,), ee[hA], jnp.float32),
                                         jnp.full((16,), ee[hB], jnp.float32))
                    hv = hsb[k, pl.ds(16 + j * 16, 16)]
                    acc_buf[k, pl.ds(16 + j * 16, 16)] = hv * mult

            pltpu.sync_copy(acc_buf, spacc.at[ids.at[1]], add=True)

        def phase(i, P, Q):
            cond1 = i + 1 < nchunks
            cond2 = i + 2 < nchunks

            def _prep_next():
                wait_ids(Q)
                issue_gather(Q)

            def _refill():
                issue_ids(cbase + i + 2, P)

            if isinstance(cond1, bool):
                if cond1:
                    _prep_next()
            else:
                pl.when(cond1)(_prep_next)
            wait_gather(P)
            compute_scatter(P)
            if isinstance(cond2, bool):
                if cond2:
                    _refill()
            else:
                pl.when(cond2)(_refill)

        issue_ids(cbase, A)
        wait_ids(A)
        issue_gather(A)
        if nchunks > 1:
            issue_ids(cbase + 1, B)

        def pair(j, carry):
            i = j * 2
            phase(i, A, B)
            phase(i + 1, B, A)
            return carry

        lax.fori_loop(0, npairs, pair, 0)
        if nchunks % 2:
            phase(nchunks - 1, A, B)
        plsc.subcore_barrier()
        pltpu.sync_copy(spacc.at[pl.ds(rows0, _RPT), :],
                        out.at[c, pl.ds(rows0, _RPT), :])

    return edge_pass


_edge_l1 = _make_edge_pass(128, 5, True)
_edge_l2 = _make_edge_pass(128, 4, False)
_edge_l3 = _make_edge_pass(64, 3, False)


def _gat_sc(x, ids2, W, a_s, a_d, b, H, C, edge_fn, feature_split):
    h = (x @ W).reshape(_N, H, C)
    hf = h.reshape(_N, H * C)
    s = (h * a_s).sum(-1)
    d = (h * a_d).sum(-1)
    M = jax.nn.leaky_relu(jnp.max(s, axis=0) + jnp.max(d, axis=0), 0.2)
    pad_hi = jnp.full((8,), 1e30, jnp.float32)
    pad8 = jnp.zeros((_N, 8), jnp.float32)
    F = H * C
    Fc = F if not feature_split else F // 2
    R = 16 + Fc

    def _dtab(dv):
        t = jnp.full((_NROWS, 16), -1e30, jnp.float32)
        return t.at[:_N, 0:8].set(dv).at[:_N, 8:16].set(0.0)

    if feature_split:
        s_r = jnp.roll(s, -4, axis=1)
        d_r = jnp.roll(d, -4, axis=1)
        M_r = jnp.roll(M, -4)
        hs0 = jnp.concatenate([s, pad8, hf[:, :Fc]], axis=1)
        hs1 = jnp.concatenate([s_r, pad8, hf[:, Fc:]], axis=1)
        dtab0 = _dtab(d)
        dtab1 = _dtab(d_r)
        m16 = jnp.stack([jnp.concatenate([M, pad_hi]),
                         jnp.concatenate([M_r, pad_hi])])
    else:
        hs0 = jnp.concatenate([s, pad8, hf], axis=1)
        hs1 = hs0
        dtab0 = _dtab(d)
        dtab1 = dtab0
        m16 = jnp.stack([jnp.concatenate([M, pad_hi])] * 2)
    zeros = jnp.zeros((_NROWS, R), jnp.float32)
    acc = edge_fn(ids2, hs0, hs1, dtab0, dtab1, m16, zeros)
    if feature_split:
        den = acc[0, :_N, 0:8]
        num = jnp.concatenate([acc[0, :_N, 16:], acc[1, :_N, 16:]], axis=1)
    else:
        den = acc[0, :_N, 0:8] + acc[1, :_N, 0:8]
        num = acc[0, :_N, 16:] + acc[1, :_N, 16:]
    ee_self = jnp.exp(jax.nn.leaky_relu(s + d, 0.2) - M[None, :])
    den = den + ee_self
    num = num.reshape(_N, H, C) + h * ee_self[:, :, None]
    out = num / (den + 1e-16)[:, :, None]
    return out.reshape(_N, H * C) + b


def _head_body(pooled_ref, wc1_ref, bc1_ref, gc1_ref, bec1_ref,
               wc2_ref, bc2_ref, gc2_ref, bec2_ref, wc3_ref, bc3_ref, out_ref):
    pooled = pooled_ref[...]
    c = pooled @ wc1_ref[...] + bc1_ref[...]
    mu = jnp.mean(c, axis=0, keepdims=True)
    var = jnp.mean((c - mu) ** 2, axis=0, keepdims=True)
    c = gc1_ref[...] * (c - mu) * jax.lax.rsqrt(var + _EPS) + bec1_ref[...]
    c = jax.nn.relu(c)
    c = c @ wc2_ref[...] + bc2_ref[...]
    mu = jnp.mean(c, axis=0, keepdims=True)
    var = jnp.mean((c - mu) ** 2, axis=0, keepdims=True)
    c = gc2_ref[...] * (c - mu) * jax.lax.rsqrt(var + _EPS) + bec2_ref[...]
    c = jax.nn.relu(c)
    out_ref[...] = c @ wc3_ref[...] + bc3_ref[...]


def _head(pooled, p):
    args = (pooled,
            p['Wc1'], p['bc1'][None, :], p['gc1'][None, :], p['bec1'][None, :],
            p['Wc2'], p['bc2'][None, :], p['gc2'][None, :], p['bec2'][None, :],
            p['Wc3'], p['bc3'][None, :])
    return pl.pallas_call(
        _head_body,
        out_shape=jax.ShapeDtypeStruct((_G, 2), jnp.float32),
    )(*args)


def kernel(x, edge_index, batch, params):
    p = params
    pad = _E_PAD - _E
    src_p = jnp.concatenate([edge_index[0], jnp.zeros((pad,), jnp.int32)])
    dst_p = jnp.concatenate([edge_index[1], jnp.full((pad,), _N, jnp.int32)])
    ids2 = jnp.stack([src_p.reshape(-1, _K), dst_p.reshape(-1, _K)], axis=1)
    h = jax.nn.relu(_bn(x @ p['W0'] + p['b0'], p['g0'], p['be0']))
    h = jax.nn.relu(_bn(_gat_sc(h, ids2, p['W1'], p['as1'], p['ad1'], p['b1'], 8, 32, _edge_l1, True), p['g1'], p['be1']))
    h = jax.nn.relu(_bn(_gat_sc(h, ids2, p['W2'], p['as2'], p['ad2'], p['b2'], 8, 16, _edge_l2, False), p['g2'], p['be2']))
    h = jax.nn.relu(_bn(_gat_sc(h, ids2, p['W3'], p['as3'], p['ad3'], p['b3'], 8, 8, _edge_l3, False), p['g3'], p['be3']))
    gate = (jax.nn.relu(_bn(h @ p['Wg1'] + p['bg1'], p['gg'], p['beg'])) @ p['Wg2'] + p['bg2'])[:, 0]
    ge = jnp.exp(gate - jnp.max(gate))
    B1h = (batch[:, None] == jnp.arange(_G, dtype=batch.dtype)[None, :]).astype(jnp.float32)
    hi = jax.lax.Precision.HIGHEST
    gden = jnp.matmul(B1h.T, ge, precision=hi)
    w = (ge / (jnp.matmul(B1h, gden, precision=hi) + 1e-16))[:, None]
    pooled = jnp.matmul(B1h.T, w * h, precision=hi)
    return _head(pooled, p)


# speedup vs baseline: 81.2242x; 1.1319x over previous
import functools

import jax
import jax.numpy as jnp
from jax import lax
from jax.experimental import pallas as pl
from jax.experimental.pallas import tpu as pltpu
from jax.experimental.pallas import tpu_sc as plsc

_N = 10000
_E = 320000
_G = 64
_EPS = 1e-5

_K = 64
_E_PAD = 323584
_NROWS = 10112
_RPT = _NROWS // 16
_DROWS = 10016


def _bn(x, g, b):
    mu = jnp.mean(x, axis=0)
    var = jnp.mean((x - mu) ** 2, axis=0)
    return g * (x - mu) / jnp.sqrt(var + _EPS) + b


def _make_edge_pass(Fc, logC, feature_split):
    R = 16 + Fc
    et = _E_PAD // 16 if feature_split else _E_PAD // 32
    nchunks = et // _K
    npairs = nchunks // 2
    mesh = plsc.VectorSubcoreMesh(core_axis_name="c", subcore_axis_name="s")

    @functools.partial(
        pl.kernel, mesh=mesh,
        compiler_params=pltpu.CompilerParams(use_tc_tiling_on_sc=False),
        out_type=jax.ShapeDtypeStruct((2, _NROWS, R), jnp.float32),
        scratch_types=[
            pltpu.VMEM((2, _K), jnp.int32),
            pltpu.VMEM((2, _K), jnp.int32),
            pltpu.VMEM((_K, R), jnp.float32),
            pltpu.VMEM((_K, R), jnp.float32),
            pltpu.VMEM((_K, 16), jnp.float32),
            pltpu.VMEM((_K, 16), jnp.float32),
            pltpu.VMEM((_K, R), jnp.float32),
            pltpu.VMEM((16,), jnp.float32),
            pltpu.VMEM_SHARED((_NROWS, R), jnp.float32),
            pltpu.SemaphoreType.DMA,
            pltpu.SemaphoreType.DMA,
            pltpu.SemaphoreType.DMA,
            pltpu.SemaphoreType.DMA,
            pltpu.SemaphoreType.DMA,
            pltpu.SemaphoreType.DMA,
        ],
    )
    def edge_pass(srcs, dsts, hs0, hs1, dtab0, dtab1, m16, out,
                  ids_a, ids_b, hs_a, hs_b, d_a, d_b, acc_buf, m_buf, spacc,
                  isem_a, isem_b, gsem_a, gsem_b, dsem_a, dsem_b):
        c = lax.axis_index("c")
        t = lax.axis_index("s")
        rows0 = pl.multiple_of(t * _RPT, 8)
        z16 = jnp.zeros((16,), jnp.float32)

        @plsc.parallel_loop(0, _K)
        def _zero(k):
            for j in range(R // 16):
                acc_buf[k, pl.ds(j * 16, 16)] = z16

        for i in range(_RPT // _K):
            pltpu.sync_copy(
                acc_buf, spacc.at[pl.ds(pl.multiple_of(rows0 + i * _K, 8), _K), :])
        rem = _RPT % _K
        if rem:
            pltpu.sync_copy(
                acc_buf.at[pl.ds(0, rem), :],
                spacc.at[pl.ds(pl.multiple_of(rows0 + (_RPT // _K) * _K, 8), rem), :])
        pltpu.sync_copy(m16.at[c], m_buf)
        plsc.subcore_barrier()
        mvec = m_buf[...]
        mask_lo = lax.iota(jnp.int32, 16) < 8
        if feature_split:
            cbase = t * nchunks
        else:
            cbase = c * (_E_PAD // (2 * _K)) + t * nchunks

        A = (ids_a, hs_a, d_a, isem_a, gsem_a, dsem_a)
        B = (ids_b, hs_b, d_b, isem_b, gsem_b, dsem_b)

        def issue_ids(m, P):
            ids, _, _, isem, _, _ = P
            pltpu.async_copy(srcs.at[m], ids.at[0], isem)
            pltpu.async_copy(dsts.at[m], ids.at[1], isem)

        def wait_ids(P):
            ids, _, _, isem, _, _ = P
            pltpu.make_async_copy(srcs.at[0], ids.at[0], isem).wait()
            pltpu.make_async_copy(dsts.at[0], ids.at[1], isem).wait()

        def issue_gather(P):
            ids, hsb, db, _, gsem, dsem = P

            @pl.when(c == 0)
            def _():
                pltpu.async_copy(hs0.at[ids.at[0]], hsb, gsem)
                pltpu.async_copy(dtab0.at[ids.at[1]], db, dsem)

            @pl.when(c == 1)
            def _():
                pltpu.async_copy(hs1.at[ids.at[0]], hsb, gsem)
                pltpu.async_copy(dtab1.at[ids.at[1]], db, dsem)

        def wait_gather(P):
            ids, hsb, db, _, gsem, dsem = P
            pltpu.make_async_copy(hs0.at[ids.at[0]], hsb, gsem).wait()
            pltpu.make_async_copy(dtab0.at[ids.at[1]], db, dsem).wait()

        def compute_scatter(P):
            ids, hsb, db, _, _, _ = P

            @plsc.parallel_loop(0, _K, unroll=4)
            def _edge(k):
                sv = hsb[k, pl.ds(0, 16)]
                e16 = sv + db[k, pl.ds(0, 16)]
                e16 = jnp.where(e16 >= 0.0, e16, 0.2 * e16)
                ee = jnp.exp(e16 - mvec)
                acc_buf[k, pl.ds(0, 16)] = ee
                if logC >= 4:
                    cpH = (1 << logC) // 16
                    for hh in range(Fc >> logC):
                        mult = jnp.full((16,), ee[hh], jnp.float32)
                        for jj in range(cpH):
                            j = hh * cpH + jj
                            hv = hsb[k, pl.ds(16 + j * 16, 16)]
                            acc_buf[k, pl.ds(16 + j * 16, 16)] = hv * mult
                else:
                    for j in range(Fc // 16):
                        hA = (j * 16) >> logC
                        hB = (j * 16 + 8) >> logC
                        mult = jnp.where(mask_lo,
                                         jnp.full((16,), ee[hA], jnp.float32),
                                         jnp.full((16,), ee[hB], jnp.float32))
                        hv = hsb[k, pl.ds(16 + j * 16, 16)]
                        acc_buf[k, pl.ds(16 + j * 16, 16)] = hv * mult

            pltpu.sync_copy(acc_buf, spacc.at[ids.at[1]], add=True)

        def phase(i, P, Q):
            cond1 = i + 1 < nchunks
            cond2 = i + 2 < nchunks

            def _prep_next():
                wait_ids(Q)
                issue_gather(Q)

            def _refill():
                issue_ids(cbase + i + 2, P)

            if isinstance(cond1, bool):
                if cond1:
                    _prep_next()
            else:
                pl.when(cond1)(_prep_next)
            wait_gather(P)
            compute_scatter(P)
            if isinstance(cond2, bool):
                if cond2:
                    _refill()
            else:
                pl.when(cond2)(_refill)

        issue_ids(cbase, A)
        wait_ids(A)
        issue_gather(A)
        if nchunks > 1:
            issue_ids(cbase + 1, B)

        def pair(j, carry):
            i = j * 2
            phase(i, A, B)
            phase(i + 1, B, A)
            return carry

        lax.fori_loop(0, npairs, pair, 0)
        if nchunks % 2:
            phase(nchunks - 1, A, B)
        plsc.subcore_barrier()
        pltpu.sync_copy(spacc.at[pl.ds(rows0, _RPT), :],
                        out.at[c, pl.ds(rows0, _RPT), :])

    return edge_pass


_edge_l1 = _make_edge_pass(128, 5, True)
_edge_l2 = _make_edge_pass(128, 4, False)
_edge_l3 = _make_edge_pass(64, 3, False)


def _gat_sc(x, srcs, dsts, W, a_s, a_d, b, H, C, edge_fn, feature_split):
    h = (x @ W).reshape(_N, H, C)
    hf = h.reshape(_N, H * C)
    s = (h * a_s).sum(-1)
    d = (h * a_d).sum(-1)
    M = jax.nn.leaky_relu(jnp.max(s, axis=0) + jnp.max(d, axis=0), 0.2)
    pad_hi = jnp.full((8,), 1e30, jnp.float32)
    pad8 = jnp.zeros((_N, 8), jnp.float32)
    F = H * C
    Fc = F if not feature_split else F // 2
    R = 16 + Fc

    def _dtab(dv):
        return jnp.pad(jnp.concatenate([dv, pad8], axis=1),
                       ((0, _DROWS - _N), (0, 0)))

    if feature_split:
        s_r = jnp.roll(s, -4, axis=1)
        d_r = jnp.roll(d, -4, axis=1)
        M_r = jnp.roll(M, -4)
        hs0 = jnp.concatenate([s, pad8, hf[:, :Fc]], axis=1)
        hs1 = jnp.concatenate([s_r, pad8, hf[:, Fc:]], axis=1)
        dtab0 = _dtab(d)
        dtab1 = _dtab(d_r)
        m16 = jnp.stack([jnp.concatenate([M, pad_hi]),
                         jnp.concatenate([M_r, pad_hi])])
    else:
        hs0 = jnp.concatenate([s, pad8, hf], axis=1)
        hs1 = hs0
        dtab0 = _dtab(d)
        dtab1 = dtab0
        m16 = jnp.stack([jnp.concatenate([M, pad_hi])] * 2)
    acc = edge_fn(srcs, dsts, hs0, hs1, dtab0, dtab1, m16)
    if feature_split:
        den = acc[0, :_N, 0:8]
        num = jnp.concatenate([acc[0, :_N, 16:], acc[1, :_N, 16:]], axis=1)
    else:
        den = acc[0, :_N, 0:8] + acc[1, :_N, 0:8]
        num = acc[0, :_N, 16:] + acc[1, :_N, 16:]
    ee_self = jnp.exp(jax.nn.leaky_relu(s + d, 0.2) - M[None, :])
    den = den + ee_self
    num = num.reshape(_N, H, C) + h * ee_self[:, :, None]
    out = num / (den + 1e-16)[:, :, None]
    return out.reshape(_N, H * C) + b


def _head_body(pooled_ref, wc1_ref, bc1_ref, gc1_ref, bec1_ref,
               wc2_ref, bc2_ref, gc2_ref, bec2_ref, wc3_ref, bc3_ref, out_ref):
    pooled = pooled_ref[...]
    c = pooled @ wc1_ref[...] + bc1_ref[...]
    mu = jnp.mean(c, axis=0, keepdims=True)
    var = jnp.mean((c - mu) ** 2, axis=0, keepdims=True)
    c = gc1_ref[...] * (c - mu) * jax.lax.rsqrt(var + _EPS) + bec1_ref[...]
    c = jax.nn.relu(c)
    c = c @ wc2_ref[...] + bc2_ref[...]
    mu = jnp.mean(c, axis=0, keepdims=True)
    var = jnp.mean((c - mu) ** 2, axis=0, keepdims=True)
    c = gc2_ref[...] * (c - mu) * jax.lax.rsqrt(var + _EPS) + bec2_ref[...]
    c = jax.nn.relu(c)
    out_ref[...] = c @ wc3_ref[...] + bc3_ref[...]


def _head(pooled, p):
    args = (pooled,
            p['Wc1'], p['bc1'][None, :], p['gc1'][None, :], p['bec1'][None, :],
            p['Wc2'], p['bc2'][None, :], p['gc2'][None, :], p['bec2'][None, :],
            p['Wc3'], p['bc3'][None, :])
    return pl.pallas_call(
        _head_body,
        out_shape=jax.ShapeDtypeStruct((_G, 2), jnp.float32),
    )(*args)


def kernel(x, edge_index, batch, params):
    p = params
    pad = _E_PAD - _E
    srcs = jnp.concatenate([edge_index[0],
                            jnp.zeros((pad,), jnp.int32)]).reshape(-1, _K)
    dsts = jnp.concatenate([edge_index[1],
                            jnp.full((pad,), _N, jnp.int32)]).reshape(-1, _K)
    h = jax.nn.relu(_bn(x @ p['W0'] + p['b0'], p['g0'], p['be0']))
    h = jax.nn.relu(_bn(_gat_sc(h, srcs, dsts, p['W1'], p['as1'], p['ad1'], p['b1'], 8, 32, _edge_l1, True), p['g1'], p['be1']))
    h = jax.nn.relu(_bn(_gat_sc(h, srcs, dsts, p['W2'], p['as2'], p['ad2'], p['b2'], 8, 16, _edge_l2, False), p['g2'], p['be2']))
    h = jax.nn.relu(_bn(_gat_sc(h, srcs, dsts, p['W3'], p['as3'], p['ad3'], p['b3'], 8, 8, _edge_l3, False), p['g3'], p['be3']))
    gate = (jax.nn.relu(_bn(h @ p['Wg1'] + p['bg1'], p['gg'], p['beg'])) @ p['Wg2'] + p['bg2'])[:, 0]
    ge = jnp.exp(gate - jnp.max(gate))
    B1h = (batch[:, None] == jnp.arange(_G, dtype=batch.dtype)[None, :]).astype(jnp.float32)
    hi = jax.lax.Precision.HIGHEST
    gden = jnp.matmul(B1h.T, ge, precision=hi)
    w = (ge / (jnp.matmul(B1h, gden, precision=hi) + 1e-16))[:, None]
    pooled = jnp.matmul(B1h.T, w * h, precision=hi)
    return _head(pooled, p)
